# baseline (speedup 1.0000x reference)
import numpy as np
import jax
import jax.numpy as jnp
from jax import lax
from jax.experimental import pallas as pl
from jax.experimental.pallas import tpu as pltpu
from jax.experimental.pallas import tpu_sc as plsc

_NB_CLASSES = 1000
_N_ROWS = 16384
_SMOOTH = np.float32(0.1)
_LOW = _SMOOTH / np.float32(_NB_CLASSES)
_HOT = (np.float32(1.0) - _SMOOTH) + _LOW

_NC = 2
_NS = 16
_NW = _NC * _NS
_ROWS_PER_W = _N_ROWS // _NW
_R = 64
_CHUNK = _R * _NB_CLASSES
_NCHUNK = _ROWS_PER_W // _R
_NBUF = 2
_G = _R // 16


def _onehot_body(x_hbm, out_hbm, x_v, buf0, buf1, sem0, sem1):
    wid = lax.axis_index("s") * _NC + lax.axis_index("c")
    base_row = wid * _ROWS_PER_W
    pltpu.sync_copy(x_hbm.at[pl.ds(base_row, _ROWS_PER_W)], x_v)

    lane = lax.iota(jnp.int32, 16)
    low = jnp.full((16,), _LOW, jnp.float32)
    hot = jnp.full((16,), _HOT, jnp.float32)

    for buf in (buf0, buf1):
        def _fill(i, _, buf=buf):
            buf[pl.ds(pl.multiple_of(i * 16, 16), 16)] = low
            return 0
        lax.fori_loop(0, _CHUNK // 16, _fill, 0)

    bufs = (buf0, buf1)
    sems = (sem0, sem1)
    pending = [None] * _NBUF
    for chunk in range(_NCHUNK):
        b = chunk % _NBUF
        buf = bufs[b]
        if pending[b] is not None:
            pending[b].wait()
            prev = chunk - _NBUF
            for g in range(_G):
                cols = x_v[pl.ds(prev * _R + g * 16, 16)]
                flat = (lane + g * 16) * _NB_CLASSES + cols
                plsc.store_scatter(buf, [flat], low)
        for g in range(_G):
            cols = x_v[pl.ds(chunk * _R + g * 16, 16)]
            flat = (lane + g * 16) * _NB_CLASSES + cols
            plsc.store_scatter(buf, [flat], hot)
        pending[b] = pltpu.async_copy(
            buf,
            out_hbm.at[pl.ds((base_row + chunk * _R) * _NB_CLASSES, _CHUNK)],
            sems[b])
    for b in range(_NBUF):
        if pending[b] is not None:
            pending[b].wait()


def kernel(x):
    x = x.astype(jnp.int32)
    run = pl.kernel(
        _onehot_body,
        out_type=jax.ShapeDtypeStruct((_N_ROWS * _NB_CLASSES,), jnp.float32),
        mesh=plsc.VectorSubcoreMesh(core_axis_name="c", subcore_axis_name="s"),
        compiler_params=pltpu.CompilerParams(needs_layout_passes=False),
        scratch_types=[
            pltpu.VMEM((_ROWS_PER_W,), jnp.int32),
            pltpu.VMEM((_CHUNK,), jnp.float32),
            pltpu.VMEM((_CHUNK,), jnp.float32),
            pltpu.SemaphoreType.DMA,
            pltpu.SemaphoreType.DMA,
        ],
    )
    return run(x).reshape(_N_ROWS, _NB_CLASSES)


# speedup vs baseline: 2.2812x; 2.2812x over previous
import numpy as np
import jax
import jax.numpy as jnp
from jax import lax
from jax.experimental import pallas as pl
from jax.experimental.pallas import tpu as pltpu

_NB_CLASSES = 1000
_N_ROWS = 16384
_SMOOTH = np.float32(0.1)
_LOW = _SMOOTH / np.float32(_NB_CLASSES)
_HOT = (np.float32(1.0) - _SMOOTH) + _LOW

_BR = 512
_GRID = _N_ROWS // _BR


def _body(x_ref, o_ref):
    xv = x_ref[0].reshape(_BR, 1)
    col = lax.broadcasted_iota(jnp.int32, (_BR, _NB_CLASSES), 1)
    o_ref[...] = jnp.where(col == xv, _HOT, _LOW)


def kernel(x):
    x3 = x.astype(jnp.int32).reshape(_GRID, 1, _BR)
    return pl.pallas_call(
        _body,
        grid=(_GRID,),
        in_specs=[pl.BlockSpec((1, 1, _BR), lambda i: (i, 0, 0))],
        out_specs=pl.BlockSpec((_BR, _NB_CLASSES), lambda i: (i, 0)),
        out_shape=jax.ShapeDtypeStruct((_N_ROWS, _NB_CLASSES), jnp.float32),
        compiler_params=pltpu.CompilerParams(
            dimension_semantics=("arbitrary",),
        ),
    )(x3)


# speedup vs baseline: 2.3689x; 1.0385x over previous
import numpy as np
import jax
import jax.numpy as jnp
from jax import lax
from jax.experimental import pallas as pl
from jax.experimental.pallas import tpu as pltpu

_NB_CLASSES = 1000
_N_ROWS = 16384
_SMOOTH = np.float32(0.1)
_LOW = _SMOOTH / np.float32(_NB_CLASSES)
_HOT = (np.float32(1.0) - _SMOOTH) + _LOW

_BR = 2048
_GRID = _N_ROWS // _BR


def _body(x_ref, o_ref):
    xv = x_ref[0].reshape(_BR, 1)
    col = lax.broadcasted_iota(jnp.int32, (_BR, _NB_CLASSES), 1)
    o_ref[...] = jnp.where(col == xv, _HOT, _LOW)


def kernel(x):
    x3 = x.astype(jnp.int32).reshape(_GRID, 1, _BR)
    return pl.pallas_call(
        _body,
        grid=(_GRID,),
        in_specs=[pl.BlockSpec((1, 1, _BR), lambda i: (i, 0, 0))],
        out_specs=pl.BlockSpec((_BR, _NB_CLASSES), lambda i: (i, 0)),
        out_shape=jax.ShapeDtypeStruct((_N_ROWS, _NB_CLASSES), jnp.float32),
        compiler_params=pltpu.CompilerParams(
            dimension_semantics=("arbitrary",),
        ),
    )(x3)


# speedup vs baseline: 2.3799x; 1.0047x over previous
import numpy as np
import jax
import jax.numpy as jnp
from jax import lax
from jax.experimental import pallas as pl
from jax.experimental.pallas import tpu as pltpu

_NB_CLASSES = 1000
_N_ROWS = 16384
_SMOOTH = np.float32(0.1)
_LOW = _SMOOTH / np.float32(_NB_CLASSES)
_HOT = (np.float32(1.0) - _SMOOTH) + _LOW

_BR = 512
_NCHUNK = _N_ROWS // _BR
_NBUF = 8


def _body(x_ref, o_ref, *scratch):
    bufs, sems = scratch[:_NBUF], scratch[_NBUF:]
    col = lax.broadcasted_iota(jnp.int32, (_BR, _NB_CLASSES), 1)
    pending = [None] * _NBUF
    for chunk in range(_NCHUNK):
        b = chunk % _NBUF
        if pending[b] is not None:
            pending[b].wait()
        xv = x_ref[chunk].reshape(_BR, 1)
        bufs[b][...] = jnp.where(col == xv, _HOT, _LOW)
        cp = pltpu.make_async_copy(
            bufs[b], o_ref.at[pl.ds(chunk * _BR, _BR)], sems[b])
        cp.start()
        pending[b] = cp
    for b in range(_NBUF):
        if pending[b] is not None:
            pending[b].wait()


def kernel(x):
    x3 = x.astype(jnp.int32).reshape(_NCHUNK, 1, _BR)
    return pl.pallas_call(
        _body,
        in_specs=[pl.BlockSpec(memory_space=pltpu.VMEM)],
        out_specs=pl.BlockSpec(memory_space=pl.ANY),
        out_shape=jax.ShapeDtypeStruct((_N_ROWS, _NB_CLASSES), jnp.float32),
        scratch_shapes=(
            [pltpu.VMEM((_BR, _NB_CLASSES), jnp.float32)] * _NBUF
            + [pltpu.SemaphoreType.DMA] * _NBUF
        ),
    )(x3)


# speedup vs baseline: 9.2710x; 3.8955x over previous
import numpy as np
import jax
import jax.numpy as jnp
from jax import lax
from jax.experimental import pallas as pl
from jax.experimental.pallas import tpu as pltpu

_NB_CLASSES = 1000
_N_ROWS = 16384
_SMOOTH = np.float32(0.1)
_LOW = _SMOOTH / np.float32(_NB_CLASSES)
_HOT = (np.float32(1.0) - _SMOOTH) + _LOW

_BI = 1024
_GRID = _N_ROWS // _BI


def _body(x_ref, o_ref):
    xv = x_ref[0]
    cls = lax.broadcasted_iota(jnp.int32, (_NB_CLASSES, _BI), 0)
    o_ref[...] = jnp.where(cls == xv, _HOT, _LOW)


def kernel(x):
    x3 = x.astype(jnp.int32).reshape(_GRID, 1, _BI)
    out_t = pl.pallas_call(
        _body,
        grid=(_GRID,),
        in_specs=[pl.BlockSpec((1, 1, _BI), lambda i: (i, 0, 0))],
        out_specs=pl.BlockSpec((_NB_CLASSES, _BI), lambda i: (0, i)),
        out_shape=jax.ShapeDtypeStruct((_NB_CLASSES, _N_ROWS), jnp.float32),
        compiler_params=pltpu.CompilerParams(
            dimension_semantics=("arbitrary",),
        ),
    )(x3)
    return out_t.T
